# baseline (device time: 66497 ns/iter reference)
import jax
import jax.numpy as jnp
from jax import lax
from jax.experimental import pallas as pl
from jax.experimental.pallas import tpu as pltpu

NY = 4
NZ = 4
K1 = 2
L1 = 1
K2 = 2


def kernel(x):
    m, n = x.shape
    nc = n // NZ
    rm = m // K1

    def body(x_ref, out_ref, p_in, p_out, s_in, s_out, g_buf,
             yp_send, yp_recv, ys_send, ys_recv,
             r_send, r_recv, l_send, l_recv):
        my_x = lax.axis_index("x")
        my_y = lax.axis_index("y")
        my_z = lax.axis_index("z")
        yl = jnp.maximum(my_y - 1, 0)
        yr = jnp.minimum(my_y + 1, NY - 1)
        zl = jnp.maximum(my_z - 1, 0)
        zr = jnp.minimum(my_z + 1, NZ - 1)
        is0 = my_y == 0
        is1 = my_y == 1
        is2 = my_y == 2
        is3 = my_y == 3
        col0 = my_z * nc
        cols = pl.ds(col0, nc)

        barrier_sem = pltpu.get_barrier_semaphore()
        for dev in (
            (my_x, (my_y - 1) % NY, my_z),
            (my_x, (my_y + 1) % NY, my_z),
            (my_x, my_y, (my_z - 1) % NZ),
            (my_x, my_y, (my_z + 1) % NZ),
        ):
            pl.semaphore_signal(
                barrier_sem, inc=1,
                device_id=dev, device_id_type=pl.DeviceIdType.MESH,
            )
        pl.semaphore_wait(barrier_sem, 4)

        def ypd(c, dev):
            return pltpu.make_async_remote_copy(
                src_ref=p_out.at[c], dst_ref=p_in.at[c],
                send_sem=yp_send.at[c], recv_sem=yp_recv.at[c],
                device_id=(my_x, dev, my_z),
                device_id_type=pl.DeviceIdType.MESH,
            )

        def ysd(c, dev):
            return pltpu.make_async_remote_copy(
                src_ref=s_out.at[c], dst_ref=s_in.at[c],
                send_sem=ys_send.at[c], recv_sem=ys_recv.at[c],
                device_id=(my_x, dev, my_z),
                device_id_type=pl.DeviceIdType.MESH,
            )

        def zflow(sub, slot, send_sems, recv_sems, d_idx, dev):
            gslice = pl.ds(slot * m + sub * rm, rm)
            return pltpu.make_async_remote_copy(
                src_ref=g_buf.at[gslice], dst_ref=g_buf.at[gslice],
                send_sem=send_sems.at[d_idx, sub],
                recv_sem=recv_sems.at[d_idx, sub],
                device_id=(my_x, my_y, dev),
                device_id_type=pl.DeviceIdType.MESH,
            )

        for i in range(K1 + L1):
            if i < K1:
                a = i
                rows_a = pl.ds(a * rm, rm)

                @pl.when(is0)
                def _():
                    p_out[a, :, :] = x_ref[rows_a, cols]
                    ypd(a, yr).start()

                @pl.when(is3)
                def _():
                    s_out[a, :, :] = x_ref[rows_a, cols]
                    ysd(a, yl).start()

                @pl.when(is1)
                def _():
                    ypd(a, yl).wait_recv()
                    p_out[a, :, :] = p_in[a, :, :] + x_ref[rows_a, cols]
                    ypd(a, yr).start()

                @pl.when(is2)
                def _():
                    ysd(a, yr).wait_recv()
                    s_out[a, :, :] = s_in[a, :, :] + x_ref[rows_a, cols]
                    ysd(a, yl).start()

            if i >= L1:
                b = i - L1
                rows_b = pl.ds(b * rm, rm)
                grows_b = pl.ds(my_z * m + b * rm, rm)

                @pl.when(is0)
                def _():
                    ysd(b, yr).wait_recv()
                    g_buf[grows_b, :] = x_ref[rows_b, cols] + s_in[b, :, :]

                @pl.when(is3)
                def _():
                    ypd(b, yl).wait_recv()
                    g_buf[grows_b, :] = x_ref[rows_b, cols] + p_in[b, :, :]

                @pl.when(is1)
                def _():
                    ysd(b, yr).wait_recv()
                    s_out[b, :, :] = s_in[b, :, :] + x_ref[rows_b, cols]
                    ysd(b, yl).start()
                    g_buf[grows_b, :] = (
                        x_ref[rows_b, cols] + p_in[b, :, :] + s_in[b, :, :]
                    )

                @pl.when(is2)
                def _():
                    ypd(b, yl).wait_recv()
                    p_out[b, :, :] = p_in[b, :, :] + x_ref[rows_b, cols]
                    ypd(b, yr).start()
                    g_buf[grows_b, :] = (
                        x_ref[rows_b, cols] + p_in[b, :, :] + s_in[b, :, :]
                    )

                @pl.when(my_z < NZ - 1)
                def _():
                    zflow(b, my_z, r_send, r_recv, 0, zr).start()

                @pl.when(my_z > 0)
                def _():
                    zflow(b, my_z, l_send, l_recv, 0, zl).start()

                out_ref[rows_b, cols] = g_buf[grows_b, :]

        for d in (1, 2, 3):
            for k2 in range(K2):
                rows_k = pl.ds(k2 * rm, rm)

                @pl.when(my_z >= d)
                def _():
                    slot = my_z - d
                    zflow(k2, slot, r_send, r_recv, d - 1, zl).wait_recv()

                if d < 3:
                    @pl.when((my_z >= d) & (my_z < NZ - 1))
                    def _():
                        zflow(k2, my_z - d, r_send, r_recv, d, zr).start()

                @pl.when(my_z >= d)
                def _():
                    slot = my_z - d
                    out_ref[rows_k, pl.ds(slot * nc, nc)] = g_buf[
                        pl.ds(slot * m + k2 * rm, rm), :
                    ]

                @pl.when(my_z + d <= NZ - 1)
                def _():
                    slot = my_z + d
                    zflow(k2, slot, l_send, l_recv, d - 1, zr).wait_recv()

                if d < 3:
                    @pl.when((my_z + d <= NZ - 1) & (my_z > 0))
                    def _():
                        zflow(k2, my_z + d, l_send, l_recv, d, zl).start()

                @pl.when(my_z + d <= NZ - 1)
                def _():
                    slot = my_z + d
                    out_ref[rows_k, pl.ds(slot * nc, nc)] = g_buf[
                        pl.ds(slot * m + k2 * rm, rm), :
                    ]

        for c in range(K1):
            @pl.when(my_y < NY - 1)
            def _():
                ypd(c, yr).wait_send()

            @pl.when(my_y > 0)
            def _():
                ysd(c, yl).wait_send()

        for k2 in range(K2):
            @pl.when(my_z < NZ - 1)
            def _():
                zflow(k2, my_z, r_send, r_recv, 0, zr).wait_send()

            @pl.when((my_z >= 1) & (my_z < NZ - 1))
            def _():
                zflow(k2, my_z - 1, r_send, r_recv, 1, zr).wait_send()

            @pl.when(my_z == 2)
            def _():
                zflow(k2, my_z - 2, r_send, r_recv, 2, zr).wait_send()

            @pl.when(my_z > 0)
            def _():
                zflow(k2, my_z, l_send, l_recv, 0, zl).wait_send()

            @pl.when((my_z <= NZ - 2) & (my_z > 0))
            def _():
                zflow(k2, my_z + 1, l_send, l_recv, 1, zl).wait_send()

            @pl.when(my_z == 1)
            def _():
                zflow(k2, my_z + 2, l_send, l_recv, 2, zl).wait_send()

    return pl.pallas_call(
        body,
        out_shape=jax.ShapeDtypeStruct((m, n), x.dtype),
        in_specs=[pl.BlockSpec(memory_space=pltpu.VMEM)],
        out_specs=pl.BlockSpec(memory_space=pltpu.VMEM),
        scratch_shapes=[
            pltpu.VMEM((K1, rm, nc), x.dtype),
            pltpu.VMEM((K1, rm, nc), x.dtype),
            pltpu.VMEM((K1, rm, nc), x.dtype),
            pltpu.VMEM((K1, rm, nc), x.dtype),
            pltpu.VMEM((NZ * m, nc), x.dtype),
            pltpu.SemaphoreType.DMA((K1,)),
            pltpu.SemaphoreType.DMA((K1,)),
            pltpu.SemaphoreType.DMA((K1,)),
            pltpu.SemaphoreType.DMA((K1,)),
            pltpu.SemaphoreType.DMA((3, K2)),
            pltpu.SemaphoreType.DMA((3, K2)),
            pltpu.SemaphoreType.DMA((3, K2)),
            pltpu.SemaphoreType.DMA((3, K2)),
        ],
        compiler_params=pltpu.CompilerParams(collective_id=0),
    )(x)


# device time: 66483 ns/iter; 1.0002x vs baseline; 1.0002x over previous
import jax
import jax.numpy as jnp
from jax import lax
from jax.experimental import pallas as pl
from jax.experimental.pallas import tpu as pltpu

NY = 4
NZ = 4
K1 = 2
L1 = 1
K2 = 2


def kernel(x):
    m, n = x.shape
    nc = n // NZ
    rm = m // K1

    def body(x_ref, out_ref, p_in, p_out, s_in, s_out,
             yp_send, yp_recv, ys_send, ys_recv,
             r_send, r_recv, l_send, l_recv):
        my_x = lax.axis_index("x")
        my_y = lax.axis_index("y")
        my_z = lax.axis_index("z")
        yl = jnp.maximum(my_y - 1, 0)
        yr = jnp.minimum(my_y + 1, NY - 1)
        zl = jnp.maximum(my_z - 1, 0)
        zr = jnp.minimum(my_z + 1, NZ - 1)
        is0 = my_y == 0
        is1 = my_y == 1
        is2 = my_y == 2
        is3 = my_y == 3
        col0 = my_z * nc
        cols = pl.ds(col0, nc)

        barrier_sem = pltpu.get_barrier_semaphore()
        for dev in (
            (my_x, (my_y - 1) % NY, my_z),
            (my_x, (my_y + 1) % NY, my_z),
            (my_x, my_y, (my_z - 1) % NZ),
            (my_x, my_y, (my_z + 1) % NZ),
        ):
            pl.semaphore_signal(
                barrier_sem, inc=1,
                device_id=dev, device_id_type=pl.DeviceIdType.MESH,
            )
        pl.semaphore_wait(barrier_sem, 4)

        def ypd(c, dev):
            return pltpu.make_async_remote_copy(
                src_ref=p_out.at[c], dst_ref=p_in.at[c],
                send_sem=yp_send.at[c], recv_sem=yp_recv.at[c],
                device_id=(my_x, dev, my_z),
                device_id_type=pl.DeviceIdType.MESH,
            )

        def ysd(c, dev):
            return pltpu.make_async_remote_copy(
                src_ref=s_out.at[c], dst_ref=s_in.at[c],
                send_sem=ys_send.at[c], recv_sem=ys_recv.at[c],
                device_id=(my_x, dev, my_z),
                device_id_type=pl.DeviceIdType.MESH,
            )

        def zflow(sub, coff, send_sems, recv_sems, d_idx, dev):
            rows = pl.ds(sub * rm, rm)
            c = pl.ds(coff, nc)
            return pltpu.make_async_remote_copy(
                src_ref=out_ref.at[rows, c], dst_ref=out_ref.at[rows, c],
                send_sem=send_sems.at[d_idx, sub],
                recv_sem=recv_sems.at[d_idx, sub],
                device_id=(my_x, my_y, dev),
                device_id_type=pl.DeviceIdType.MESH,
            )

        for i in range(K1 + L1):
            if i < K1:
                a = i
                rows_a = pl.ds(a * rm, rm)

                @pl.when(is0)
                def _():
                    p_out[a, :, :] = x_ref[rows_a, cols]
                    ypd(a, yr).start()

                @pl.when(is3)
                def _():
                    s_out[a, :, :] = x_ref[rows_a, cols]
                    ysd(a, yl).start()

                @pl.when(is1)
                def _():
                    ypd(a, yl).wait_recv()
                    p_out[a, :, :] = p_in[a, :, :] + x_ref[rows_a, cols]
                    ypd(a, yr).start()

                @pl.when(is2)
                def _():
                    ysd(a, yr).wait_recv()
                    s_out[a, :, :] = s_in[a, :, :] + x_ref[rows_a, cols]
                    ysd(a, yl).start()

            if i >= L1:
                b = i - L1
                rows_b = pl.ds(b * rm, rm)

                @pl.when(is0)
                def _():
                    ysd(b, yr).wait_recv()
                    out_ref[rows_b, cols] = x_ref[rows_b, cols] + s_in[b, :, :]

                @pl.when(is3)
                def _():
                    ypd(b, yl).wait_recv()
                    out_ref[rows_b, cols] = x_ref[rows_b, cols] + p_in[b, :, :]

                @pl.when(is1)
                def _():
                    ysd(b, yr).wait_recv()
                    s_out[b, :, :] = s_in[b, :, :] + x_ref[rows_b, cols]
                    ysd(b, yl).start()
                    out_ref[rows_b, cols] = (
                        x_ref[rows_b, cols] + p_in[b, :, :] + s_in[b, :, :]
                    )

                @pl.when(is2)
                def _():
                    ypd(b, yl).wait_recv()
                    p_out[b, :, :] = p_in[b, :, :] + x_ref[rows_b, cols]
                    ypd(b, yr).start()
                    out_ref[rows_b, cols] = (
                        x_ref[rows_b, cols] + p_in[b, :, :] + s_in[b, :, :]
                    )

                @pl.when(my_z < NZ - 1)
                def _():
                    zflow(b, col0, r_send, r_recv, 0, zr).start()

                @pl.when(my_z > 0)
                def _():
                    zflow(b, col0, l_send, l_recv, 0, zl).start()

        for d in (1, 2, 3):
            for k2 in range(K2):
                @pl.when(my_z >= d)
                def _():
                    coff = (my_z - d) * nc
                    zflow(k2, coff, r_send, r_recv, d - 1, zl).wait_recv()

                if d < 3:
                    @pl.when((my_z >= d) & (my_z < NZ - 1))
                    def _():
                        coff = (my_z - d) * nc
                        zflow(k2, coff, r_send, r_recv, d, zr).start()

                @pl.when(my_z + d <= NZ - 1)
                def _():
                    coff = (my_z + d) * nc
                    zflow(k2, coff, l_send, l_recv, d - 1, zr).wait_recv()

                if d < 3:
                    @pl.when((my_z + d <= NZ - 1) & (my_z > 0))
                    def _():
                        coff = (my_z + d) * nc
                        zflow(k2, coff, l_send, l_recv, d, zl).start()

        for c in range(K1):
            @pl.when(my_y < NY - 1)
            def _():
                ypd(c, yr).wait_send()

            @pl.when(my_y > 0)
            def _():
                ysd(c, yl).wait_send()

        for k2 in range(K2):
            @pl.when(my_z < NZ - 1)
            def _():
                zflow(k2, col0, r_send, r_recv, 0, zr).wait_send()

            @pl.when((my_z >= 1) & (my_z < NZ - 1))
            def _():
                zflow(k2, (my_z - 1) * nc, r_send, r_recv, 1, zr).wait_send()

            @pl.when(my_z == 2)
            def _():
                zflow(k2, (my_z - 2) * nc, r_send, r_recv, 2, zr).wait_send()

            @pl.when(my_z > 0)
            def _():
                zflow(k2, col0, l_send, l_recv, 0, zl).wait_send()

            @pl.when((my_z <= NZ - 2) & (my_z > 0))
            def _():
                zflow(k2, (my_z + 1) * nc, l_send, l_recv, 1, zl).wait_send()

            @pl.when(my_z == 1)
            def _():
                zflow(k2, (my_z + 2) * nc, l_send, l_recv, 2, zl).wait_send()

    return pl.pallas_call(
        body,
        out_shape=jax.ShapeDtypeStruct((m, n), x.dtype),
        in_specs=[pl.BlockSpec(memory_space=pltpu.VMEM)],
        out_specs=pl.BlockSpec(memory_space=pltpu.VMEM),
        scratch_shapes=[
            pltpu.VMEM((K1, rm, nc), x.dtype),
            pltpu.VMEM((K1, rm, nc), x.dtype),
            pltpu.VMEM((K1, rm, nc), x.dtype),
            pltpu.VMEM((K1, rm, nc), x.dtype),
            pltpu.SemaphoreType.DMA((K1,)),
            pltpu.SemaphoreType.DMA((K1,)),
            pltpu.SemaphoreType.DMA((K1,)),
            pltpu.SemaphoreType.DMA((K1,)),
            pltpu.SemaphoreType.DMA((3, K2)),
            pltpu.SemaphoreType.DMA((3, K2)),
            pltpu.SemaphoreType.DMA((3, K2)),
            pltpu.SemaphoreType.DMA((3, K2)),
        ],
        compiler_params=pltpu.CompilerParams(collective_id=0),
    )(x)


# device time: 63467 ns/iter; 1.0477x vs baseline; 1.0475x over previous
import jax
import jax.numpy as jnp
from jax import lax
from jax.experimental import pallas as pl
from jax.experimental.pallas import tpu as pltpu

NY = 4
NZ = 4
NX = 2
K1 = 2
L1 = 1
K2 = 2


def kernel(x):
    m, n = x.shape
    nc = n // NZ
    mh = m // NX
    rm = mh // K1

    def body(x_ref, out_ref, p_in, p_out, s_in, s_out,
             yp_send, yp_recv, ys_send, ys_recv,
             r_send, r_recv, l_send, l_recv, x_send, x_recv):
        my_x = lax.axis_index("x")
        my_y = lax.axis_index("y")
        my_z = lax.axis_index("z")
        yl = jnp.maximum(my_y - 1, 0)
        yr = jnp.minimum(my_y + 1, NY - 1)
        zl = jnp.maximum(my_z - 1, 0)
        zr = jnp.minimum(my_z + 1, NZ - 1)
        xp = 1 - my_x
        is0 = my_y == 0
        is1 = my_y == 1
        is2 = my_y == 2
        is3 = my_y == 3
        col0 = my_z * nc
        cols = pl.ds(col0, nc)
        r0 = my_x * mh
        pr0 = xp * mh

        barrier_sem = pltpu.get_barrier_semaphore()
        for dev in (
            (my_x, (my_y - 1) % NY, my_z),
            (my_x, (my_y + 1) % NY, my_z),
            (my_x, my_y, (my_z - 1) % NZ),
            (my_x, my_y, (my_z + 1) % NZ),
            (xp, my_y, my_z),
        ):
            pl.semaphore_signal(
                barrier_sem, inc=1,
                device_id=dev, device_id_type=pl.DeviceIdType.MESH,
            )
        pl.semaphore_wait(barrier_sem, 5)

        def ypd(c, dev):
            return pltpu.make_async_remote_copy(
                src_ref=p_out.at[c], dst_ref=p_in.at[c],
                send_sem=yp_send.at[c], recv_sem=yp_recv.at[c],
                device_id=(my_x, dev, my_z),
                device_id_type=pl.DeviceIdType.MESH,
            )

        def ysd(c, dev):
            return pltpu.make_async_remote_copy(
                src_ref=s_out.at[c], dst_ref=s_in.at[c],
                send_sem=ys_send.at[c], recv_sem=ys_recv.at[c],
                device_id=(my_x, dev, my_z),
                device_id_type=pl.DeviceIdType.MESH,
            )

        def zflow(sub, coff, send_sems, recv_sems, d_idx, dev):
            rows = pl.ds(r0 + sub * rm, rm)
            c = pl.ds(coff, nc)
            return pltpu.make_async_remote_copy(
                src_ref=out_ref.at[rows, c], dst_ref=out_ref.at[rows, c],
                send_sem=send_sems.at[d_idx, sub],
                recv_sem=recv_sems.at[d_idx, sub],
                device_id=(my_x, my_y, dev),
                device_id_type=pl.DeviceIdType.MESH,
            )

        def xflow(sub, base):
            rows = pl.ds(base + sub * rm, rm)
            return pltpu.make_async_remote_copy(
                src_ref=out_ref.at[rows], dst_ref=out_ref.at[rows],
                send_sem=x_send.at[sub], recv_sem=x_recv.at[sub],
                device_id=(xp, my_y, my_z),
                device_id_type=pl.DeviceIdType.MESH,
            )

        for i in range(K1 + L1):
            if i < K1:
                a = i
                rows_a = pl.ds(r0 + a * rm, rm)

                @pl.when(is0)
                def _():
                    p_out[a, :, :] = x_ref[rows_a, cols]
                    ypd(a, yr).start()

                @pl.when(is3)
                def _():
                    s_out[a, :, :] = x_ref[rows_a, cols]
                    ysd(a, yl).start()

                @pl.when(is1)
                def _():
                    ypd(a, yl).wait_recv()
                    p_out[a, :, :] = p_in[a, :, :] + x_ref[rows_a, cols]
                    ypd(a, yr).start()

                @pl.when(is2)
                def _():
                    ysd(a, yr).wait_recv()
                    s_out[a, :, :] = s_in[a, :, :] + x_ref[rows_a, cols]
                    ysd(a, yl).start()

            if i >= L1:
                b = i - L1
                rows_b = pl.ds(r0 + b * rm, rm)

                @pl.when(is0)
                def _():
                    ysd(b, yr).wait_recv()
                    out_ref[rows_b, cols] = x_ref[rows_b, cols] + s_in[b, :, :]

                @pl.when(is3)
                def _():
                    ypd(b, yl).wait_recv()
                    out_ref[rows_b, cols] = x_ref[rows_b, cols] + p_in[b, :, :]

                @pl.when(is1)
                def _():
                    ysd(b, yr).wait_recv()
                    s_out[b, :, :] = s_in[b, :, :] + x_ref[rows_b, cols]
                    ysd(b, yl).start()
                    out_ref[rows_b, cols] = (
                        x_ref[rows_b, cols] + p_in[b, :, :] + s_in[b, :, :]
                    )

                @pl.when(is2)
                def _():
                    ypd(b, yl).wait_recv()
                    p_out[b, :, :] = p_in[b, :, :] + x_ref[rows_b, cols]
                    ypd(b, yr).start()
                    out_ref[rows_b, cols] = (
                        x_ref[rows_b, cols] + p_in[b, :, :] + s_in[b, :, :]
                    )

                @pl.when(my_z < NZ - 1)
                def _():
                    zflow(b, col0, r_send, r_recv, 0, zr).start()

                @pl.when(my_z > 0)
                def _():
                    zflow(b, col0, l_send, l_recv, 0, zl).start()

        for d in (1, 2, 3):
            for k2 in range(K2):
                @pl.when(my_z >= d)
                def _():
                    coff = (my_z - d) * nc
                    zflow(k2, coff, r_send, r_recv, d - 1, zl).wait_recv()

                if d < 3:
                    @pl.when((my_z >= d) & (my_z < NZ - 1))
                    def _():
                        coff = (my_z - d) * nc
                        zflow(k2, coff, r_send, r_recv, d, zr).start()

                @pl.when(my_z + d <= NZ - 1)
                def _():
                    coff = (my_z + d) * nc
                    zflow(k2, coff, l_send, l_recv, d - 1, zr).wait_recv()

                if d < 3:
                    @pl.when((my_z + d <= NZ - 1) & (my_z > 0))
                    def _():
                        coff = (my_z + d) * nc
                        zflow(k2, coff, l_send, l_recv, d, zl).start()

                if d == 3:
                    xflow(k2, r0).start()

        for k2 in range(K2):
            xflow(k2, pr0).wait_recv()

        for c in range(K1):
            @pl.when(my_y < NY - 1)
            def _():
                ypd(c, yr).wait_send()

            @pl.when(my_y > 0)
            def _():
                ysd(c, yl).wait_send()

        for k2 in range(K2):
            @pl.when(my_z < NZ - 1)
            def _():
                zflow(k2, col0, r_send, r_recv, 0, zr).wait_send()

            @pl.when((my_z >= 1) & (my_z < NZ - 1))
            def _():
                zflow(k2, (my_z - 1) * nc, r_send, r_recv, 1, zr).wait_send()

            @pl.when(my_z == 2)
            def _():
                zflow(k2, (my_z - 2) * nc, r_send, r_recv, 2, zr).wait_send()

            @pl.when(my_z > 0)
            def _():
                zflow(k2, col0, l_send, l_recv, 0, zl).wait_send()

            @pl.when((my_z <= NZ - 2) & (my_z > 0))
            def _():
                zflow(k2, (my_z + 1) * nc, l_send, l_recv, 1, zl).wait_send()

            @pl.when(my_z == 1)
            def _():
                zflow(k2, (my_z + 2) * nc, l_send, l_recv, 2, zl).wait_send()

            xflow(k2, r0).wait_send()

    return pl.pallas_call(
        body,
        out_shape=jax.ShapeDtypeStruct((m, n), x.dtype),
        in_specs=[pl.BlockSpec(memory_space=pltpu.VMEM)],
        out_specs=pl.BlockSpec(memory_space=pltpu.VMEM),
        scratch_shapes=[
            pltpu.VMEM((K1, rm, nc), x.dtype),
            pltpu.VMEM((K1, rm, nc), x.dtype),
            pltpu.VMEM((K1, rm, nc), x.dtype),
            pltpu.VMEM((K1, rm, nc), x.dtype),
            pltpu.SemaphoreType.DMA((K1,)),
            pltpu.SemaphoreType.DMA((K1,)),
            pltpu.SemaphoreType.DMA((K1,)),
            pltpu.SemaphoreType.DMA((K1,)),
            pltpu.SemaphoreType.DMA((3, K2)),
            pltpu.SemaphoreType.DMA((3, K2)),
            pltpu.SemaphoreType.DMA((3, K2)),
            pltpu.SemaphoreType.DMA((3, K2)),
            pltpu.SemaphoreType.DMA((K2,)),
            pltpu.SemaphoreType.DMA((K2,)),
        ],
        compiler_params=pltpu.CompilerParams(collective_id=0),
    )(x)
